# Initial kernel scaffold; baseline (speedup 1.0000x reference)
import jax
import jax.numpy as jnp
from jax.experimental import pallas as pl


def kernel(x, edge_index, W, b):
    raise NotImplementedError("write your pallas kernel here")


# speedup vs baseline: 4.9679x; 4.9679x over previous
import functools

import jax
import jax.numpy as jnp
from jax import lax
from jax.experimental import pallas as pl
from jax.experimental.pallas import tpu as pltpu
from jax.experimental.pallas import tpu_sc as plsc

_N = 10000
_E = 320000
_D = 128
_DH = _D // 2
_NPAD = 10240
_NC = 2
_NS = 16
_NW = _NC * _NS
_EPW = _E // _NW
_EPT = _E // _NS
_CHUNK = 80
_NCHUNK = _EPT // _CHUNK
_RPT = _NPAD // _NS

_MESH = plsc.VectorSubcoreMesh(
    core_axis_name="c", subcore_axis_name="s", num_cores=_NC, num_subcores=_NS
)
_SC_PARAMS = pltpu.CompilerParams(
    needs_layout_passes=False, use_tc_tiling_on_sc=False
)


def _sc_degrees(src2, dst2):
  @functools.partial(
      pl.kernel,
      out_type=jax.ShapeDtypeStruct((_NW, 2, _NPAD), jnp.float32),
      mesh=_MESH,
      compiler_params=_SC_PARAMS,
      scratch_types=[
          pltpu.VMEM((_EPW,), jnp.int32),
          pltpu.VMEM((_EPW,), jnp.int32),
          pltpu.VMEM((_NPAD,), jnp.float32),
          pltpu.VMEM((_NPAD,), jnp.float32),
      ],
  )
  def k(src_hbm, dst_hbm, out_hbm, src_v, dst_v, hist_s, hist_d):
    c = lax.axis_index("c")
    s = lax.axis_index("s")
    wid = c * _NS + s
    zero = jnp.zeros((16,), jnp.float32)

    def zb(i, carry):
      hist_s[pl.ds(i * 16, 16)] = zero
      hist_d[pl.ds(i * 16, 16)] = zero
      return carry

    lax.fori_loop(0, _NPAD // 16, zb, 0)
    pltpu.sync_copy(src_hbm.at[wid], src_v)
    pltpu.sync_copy(dst_hbm.at[wid], dst_v)
    ones = jnp.ones((16,), jnp.float32)

    def eb(i, carry):
      plsc.addupdate_scatter(hist_s, [src_v[pl.ds(i * 16, 16)]], ones)
      plsc.addupdate_scatter(hist_d, [dst_v[pl.ds(i * 16, 16)]], ones)
      return carry

    lax.fori_loop(0, _EPW // 16, eb, 0)
    pltpu.sync_copy(hist_s, out_hbm.at[wid, 0])
    pltpu.sync_copy(hist_d, out_hbm.at[wid, 1])

  return k(src2, dst2)


def _tc_norms(deg_part):
  def body(deg_ref, out_ref):
    deg = jnp.sum(deg_ref[...], axis=0)
    out_ref[...] = lax.rsqrt(jnp.maximum(deg, 1.0))

  return pl.pallas_call(
      body,
      out_shape=jax.ShapeDtypeStruct((2, _NPAD), jnp.float32),
  )(deg_part)


def _tc_scale(x, nsrc_col):
  def body(x_ref, n_ref, o_ref):
    h = x_ref[...] * n_ref[...]
    o_ref[0, ...] = h[:, 0:_DH]
    o_ref[1, ...] = h[:, _DH:_D]

  rows = 2000
  return pl.pallas_call(
      body,
      grid=(_N // rows,),
      in_specs=[
          pl.BlockSpec((rows, _D), lambda i: (i, 0)),
          pl.BlockSpec((rows, 1), lambda i: (i, 0)),
      ],
      out_specs=pl.BlockSpec((2, rows, _DH), lambda i: (0, i, 0)),
      out_shape=jax.ShapeDtypeStruct((2, _N, _DH), jnp.float32),
  )(x, nsrc_col)


def _sc_aggregate(h0, h1, src3, dst3):
  @functools.partial(
      pl.kernel,
      out_type=jax.ShapeDtypeStruct((_NC, _NPAD, _DH), jnp.float32),
      mesh=_MESH,
      compiler_params=_SC_PARAMS,
      scratch_types=[
          pltpu.VMEM((_NCHUNK, _CHUNK), jnp.int32),
          pltpu.VMEM((_NCHUNK, _CHUNK), jnp.int32),
          pltpu.VMEM((_CHUNK, _DH), jnp.float32),
          pltpu.VMEM((_CHUNK, _DH), jnp.float32),
          pltpu.VMEM_SHARED((_NPAD, _DH), jnp.float32),
          pltpu.SemaphoreType.DMA,
      ],
  )
  def k(h0_hbm, h1_hbm, src_hbm, dst_hbm, out_hbm, src_v, dst_v, rows_v, zbuf,
        acc, sem):
    c = lax.axis_index("c")
    s = lax.axis_index("s")
    zero = jnp.zeros((16,), jnp.float32)
    nsub = _DH // 16

    def zb(i, carry):
      zbuf[i // nsub, pl.ds((i % nsub) * 16, 16)] = zero
      return carry

    lax.fori_loop(0, _CHUNK * nsub, zb, 0)
    base = s * _RPT
    for r in range(_RPT // _CHUNK):
      pltpu.sync_copy(zbuf, acc.at[pl.ds(base + r * _CHUNK, _CHUNK)])
    pltpu.sync_copy(src_hbm.at[s], src_v)
    pltpu.sync_copy(dst_hbm.at[s], dst_v)
    plsc.subcore_barrier()

    def run(h_hbm):
      def body(j, carry):
        pltpu.async_copy(h_hbm.at[src_v.at[j]], rows_v, sem).wait()
        pltpu.sync_copy(rows_v, acc.at[dst_v.at[j]], add=True)
        return carry

      lax.fori_loop(0, _NCHUNK, body, 0)

    @pl.when(c == 0)
    def _():
      run(h0_hbm)

    @pl.when(c == 1)
    def _():
      run(h1_hbm)

    plsc.subcore_barrier()
    pltpu.sync_copy(acc.at[pl.ds(base, _RPT)], out_hbm.at[c, pl.ds(base, _RPT)])

  return k(h0, h1, src3, dst3)


def _tc_final(a0, a1, nd_col, W, b2):
  def body(a0_ref, a1_ref, n_ref, w_ref, b_ref, o_ref):
    n = n_ref[...]
    o_ref[...] = (
        jnp.dot(a0_ref[...] * n, w_ref[0:_DH, :],
                preferred_element_type=jnp.float32)
        + jnp.dot(a1_ref[...] * n, w_ref[_DH:_D, :],
                  preferred_element_type=jnp.float32)
        + b_ref[...]
    )

  rows = 2048
  return pl.pallas_call(
      body,
      grid=(_NPAD // rows,),
      in_specs=[
          pl.BlockSpec((rows, _DH), lambda i: (i, 0)),
          pl.BlockSpec((rows, _DH), lambda i: (i, 0)),
          pl.BlockSpec((rows, 1), lambda i: (i, 0)),
          pl.BlockSpec((_D, _D), lambda i: (0, 0)),
          pl.BlockSpec((1, _D), lambda i: (0, 0)),
      ],
      out_specs=pl.BlockSpec((rows, _D), lambda i: (i, 0)),
      out_shape=jax.ShapeDtypeStruct((_NPAD, _D), jnp.float32),
  )(a0, a1, nd_col, W, b2)


def kernel(x, edge_index, W, b):
  src = edge_index[0]
  dst = edge_index[1]
  deg_part = _sc_degrees(src.reshape(_NW, _EPW), dst.reshape(_NW, _EPW))
  norms = _tc_norms(deg_part)
  nsrc_col = norms[0, :_N].reshape(_N, 1)
  nd_col = norms[1].reshape(_NPAD, 1)
  h2 = _tc_scale(x, nsrc_col)
  agg = _sc_aggregate(
      h2[0],
      h2[1],
      src.reshape(_NS, _NCHUNK, _CHUNK),
      dst.reshape(_NS, _NCHUNK, _CHUNK),
  )
  out = _tc_final(agg[0], agg[1], nd_col, W, b.reshape(1, _D))
  return out[:_N]
